# baseline (speedup 1.0000x reference)
import functools

import jax
import jax.numpy as jnp
from jax import lax
from jax.experimental import pallas as pl
from jax.experimental.pallas import tpu as pltpu
from jax.experimental.pallas import tpu_sc as plsc

N_USERS = 1000000
N_MOVIES = 100000
EMBED_DIM = 32
BATCH = 16384

NC = 2
NS = 16
NW = NC * NS
BPW = BATCH // NW
NCHUNK = 4
CHUNK = BPW // NCHUNK
L = 16


def _sc_body(in_hbm, user_hbm, movie_hbm, out_hbm,
             in_v, uix_v, mix_v, urows_v, mrows_v, out_v, sem):
    c = lax.axis_index("c")
    s = lax.axis_index("s")
    wid = s * NC + c
    base = wid * BPW

    pltpu.sync_copy(in_hbm.at[wid], in_v)

    iota = lax.iota(jnp.int32, L)
    for g in range(BPW // L):
        pos = 2 * L * g + 2 * iota
        u = plsc.load_gather(in_v, [pos])
        m = plsc.load_gather(in_v, [pos + 1])
        j, off = divmod(g, CHUNK // L)
        uix_v[j, pl.ds(off * L, L)] = u
        mix_v[j, pl.ds(off * L, L)] = m

    copies = []
    for j in range(NCHUNK):
        copies.append(pltpu.async_copy(
            user_hbm.at[uix_v.at[j]], urows_v.at[pl.ds(j * CHUNK, CHUNK)], sem))
        copies.append(pltpu.async_copy(
            movie_hbm.at[mix_v.at[j]], mrows_v.at[pl.ds(j * CHUNK, CHUNK)], sem))
    for cp in copies:
        cp.wait()

    def group(g, _):
        rows = g * L + iota
        acc = jnp.zeros((L,), jnp.float32)
        for d in range(EMBED_DIM):
            col = jnp.full((L,), d, jnp.int32)
            cu = plsc.load_gather(urows_v, [rows, col])
            cm = plsc.load_gather(mrows_v, [rows, col])
            acc = acc + cu * cm
        out_v[pl.ds(g * L, L)] = acc
        return _

    lax.fori_loop(0, BPW // L, group, 0)

    pltpu.sync_copy(out_v, out_hbm.at[pl.ds(base, BPW)])


def kernel(inputs, user_table, movie_table):
    inputs = jnp.reshape(inputs.astype(jnp.int32), (NW, 2 * BPW))
    mesh = plsc.VectorSubcoreMesh(core_axis_name="c", subcore_axis_name="s")
    run = functools.partial(
        pl.kernel,
        mesh=mesh,
        compiler_params=pltpu.CompilerParams(
            needs_layout_passes=False, use_tc_tiling_on_sc=False),
        out_type=jax.ShapeDtypeStruct((BATCH,), jnp.float32),
        scratch_types=[
            pltpu.VMEM((2 * BPW,), jnp.int32),
            pltpu.VMEM((NCHUNK, CHUNK), jnp.int32),
            pltpu.VMEM((NCHUNK, CHUNK), jnp.int32),
            pltpu.VMEM((BPW, EMBED_DIM), jnp.float32),
            pltpu.VMEM((BPW, EMBED_DIM), jnp.float32),
            pltpu.VMEM((BPW,), jnp.float32),
            pltpu.SemaphoreType.DMA,
        ],
    )(_sc_body)
    return run(inputs, user_table, movie_table)
